# baseline (speedup 1.0000x reference)
import dataclasses
import functools

import jax
import jax.numpy as jnp
from jax import lax
from jax.experimental import pallas as pl
from jax.experimental.pallas import tpu as pltpu
from jax.experimental.pallas import tpu_sc as plsc

_LANES = 16
_CHUNK = 64
_SEGS_PER_WORKER = 16
_NUM_WORKERS = 32


def _sc_pool(flat, cu_pad, total_tok, emb, num_sents):
    groups = emb // _LANES
    mesh = plsc.VectorSubcoreMesh(core_axis_name="core",
                                  subcore_axis_name="subcore")
    cp = pltpu.CompilerParams()
    if "needs_layout_passes" in pltpu.CompilerParams.__dataclass_fields__:
        cp = dataclasses.replace(cp, needs_layout_passes=False)

    @functools.partial(
        pl.kernel,
        out_type=jax.ShapeDtypeStruct((num_sents, emb), jnp.float32),
        mesh=mesh,
        compiler_params=cp,
        scratch_types=[
            pltpu.VMEM((_CHUNK, emb), jnp.float32),
            pltpu.VMEM((_SEGS_PER_WORKER, emb), jnp.float32),
            pltpu.VMEM((2 * _SEGS_PER_WORKER,), jnp.int32),
        ],
    )
    def pool(flat_hbm, cu_hbm, out_hbm, buf, stage, cuts_v):
        w = lax.axis_index("core") * 16 + lax.axis_index("subcore")
        seg_base = w * _SEGS_PER_WORKER
        pltpu.sync_copy(cu_hbm.at[pl.ds(seg_base, 2 * _SEGS_PER_WORKER)],
                        cuts_v)
        v_lo = cuts_v[pl.ds(0, _LANES)]
        v_hi = cuts_v[pl.ds(_LANES, _LANES)]
        lane = lax.iota(jnp.int32, _LANES)

        def extract(vec, i):
            return jnp.sum(jnp.where(lane == i, vec, 0))

        for i in range(_SEGS_PER_WORKER):
            start = extract(v_lo, i)
            end = (extract(v_lo, i + 1) if i + 1 < _LANES
                   else extract(v_hi, 0))
            pos0 = jnp.minimum((start // 8) * 8, total_tok - _CHUNK)
            nch = jnp.maximum((end - pos0 + _CHUNK - 1) // _CHUNK, 0)

            def row_body(r, acc, _buf=buf):
                return tuple(
                    acc[k] + _buf[r, pl.ds(k * _LANES, _LANES)]
                    for k in range(groups))

            def chunk_body(c, acc, _start=start, _end=end, _pos0=pos0,
                           _buf=buf):
                pos = _pos0 + c * _CHUNK
                d = jnp.minimum(pos, total_tok - _CHUNK)
                pltpu.sync_copy(flat_hbm.at[pl.ds(d, _CHUNK)], _buf)
                r_lo = jnp.maximum(_start, pos) - d
                r_hi = jnp.minimum(_end, pos + _CHUNK) - d
                return lax.fori_loop(r_lo, r_hi, row_body, acc)

            acc0 = tuple(jnp.zeros((_LANES,), jnp.float32)
                         for _ in range(groups))
            acc = lax.fori_loop(0, nch, chunk_body, acc0)
            for k in range(groups):
                stage[i, pl.ds(k * _LANES, _LANES)] = acc[k]

        pltpu.sync_copy(stage, out_hbm.at[pl.ds(seg_base, _SEGS_PER_WORKER)])

    return pool(flat, cu_pad)


def _head_body(sums_ref, inv_ref, we_ref, wc_ref, out_ref):
    mean = sums_ref[...] * inv_ref[...]
    combined = jax.lax.dot_general(
        wc_ref[...], we_ref[...], (((1,), (0,)), ((), ())),
        precision=jax.lax.Precision.HIGHEST,
        preferred_element_type=jnp.float32)
    logits = jax.lax.dot_general(
        mean, combined, (((1,), (1,)), ((), ())),
        precision=jax.lax.Precision.HIGHEST,
        preferred_element_type=jnp.float32)
    m = jnp.max(logits, axis=-1, keepdims=True)
    sh = logits - m
    lse = jnp.log(jnp.sum(jnp.exp(sh), axis=-1, keepdims=True))
    out_ref[...] = sh - lse


def kernel(flat, cu_seqlens, W_e, W_c):
    total_tok, emb = flat.shape
    num_sents = cu_seqlens.shape[0] - 1
    ntags = W_c.shape[0]

    cu = cu_seqlens.astype(jnp.int32)
    inv = 1.0 / jnp.maximum(cu[1:] - cu[:-1], 1).astype(jnp.float32)
    inv = inv.reshape(num_sents, 1)
    cu_pad = jnp.full((num_sents + 2 * _SEGS_PER_WORKER,), total_tok,
                      jnp.int32).at[:num_sents + 1].set(cu)

    sums = _sc_pool(flat, cu_pad, total_tok, emb, num_sents)

    out = pl.pallas_call(
        _head_body,
        in_specs=[
            pl.BlockSpec((num_sents, emb), lambda: (0, 0)),
            pl.BlockSpec((num_sents, 1), lambda: (0, 0)),
            pl.BlockSpec((emb, emb), lambda: (0, 0)),
            pl.BlockSpec((ntags, emb), lambda: (0, 0)),
        ],
        out_specs=pl.BlockSpec((num_sents, ntags), lambda: (0, 0)),
        out_shape=jax.ShapeDtypeStruct((num_sents, ntags), jnp.float32),
    )(sums, inv, W_e, W_c)
    return out


# speedup vs baseline: 1.6724x; 1.6724x over previous
import dataclasses
import functools

import jax
import jax.numpy as jnp
from jax import lax
from jax.experimental import pallas as pl
from jax.experimental.pallas import tpu as pltpu
from jax.experimental.pallas import tpu_sc as plsc

_LANES = 16
_CHUNK = 64
_SEGS_PER_WORKER = 16
_NUM_WORKERS = 32


def _sc_pool(flat, cu_pad, total_tok, emb, num_sents):
    groups = emb // _LANES
    mesh = plsc.VectorSubcoreMesh(core_axis_name="core",
                                  subcore_axis_name="subcore")
    cp = pltpu.CompilerParams()
    if "needs_layout_passes" in pltpu.CompilerParams.__dataclass_fields__:
        cp = dataclasses.replace(cp, needs_layout_passes=False)

    @functools.partial(
        pl.kernel,
        out_type=jax.ShapeDtypeStruct((num_sents, emb), jnp.float32),
        mesh=mesh,
        compiler_params=cp,
        scratch_types=[
            pltpu.VMEM((_CHUNK, emb), jnp.float32),
            pltpu.VMEM((_CHUNK, emb), jnp.float32),
            pltpu.VMEM((_SEGS_PER_WORKER, emb), jnp.float32),
            pltpu.VMEM((2 * _SEGS_PER_WORKER,), jnp.int32),
            pltpu.SemaphoreType.DMA,
            pltpu.SemaphoreType.DMA,
        ],
    )
    def pool(flat_hbm, cu_hbm, out_hbm, buf_a, buf_b, stage, cuts_v,
             sem_a, sem_b):
        w = lax.axis_index("core") * 16 + lax.axis_index("subcore")
        seg_base = w * _SEGS_PER_WORKER
        pltpu.sync_copy(cu_hbm.at[pl.ds(seg_base, 2 * _SEGS_PER_WORKER)],
                        cuts_v)
        v_lo = cuts_v[pl.ds(0, _LANES)]
        v_hi = cuts_v[pl.ds(_LANES, _LANES)]
        lane = lax.iota(jnp.int32, _LANES)

        def extract(j):
            return (jnp.sum(jnp.where(lane == j, v_lo, 0))
                    + jnp.sum(jnp.where(lane == (j - _LANES), v_hi, 0)))

        zero_v = jnp.zeros((_LANES,), jnp.float32)

        def zero_seg(i, _):
            for k in range(groups):
                stage[i, pl.ds(k * _LANES, _LANES)] = zero_v
            return 0

        lax.fori_loop(0, _SEGS_PER_WORKER, zero_seg, 0)

        start_w = extract(0)
        end_w = extract(_SEGS_PER_WORKER)
        pos0 = jnp.minimum((start_w // 8) * 8, total_tok - _CHUNK)
        nch = jnp.maximum((end_w - pos0 + _CHUNK - 1) // _CHUNK, 0)
        npair = (nch + 1) // 2

        def issue(c, buf, sem):
            d = jnp.minimum(pos0 + c * _CHUNK, total_tok - _CHUNK)
            return pltpu.async_copy(flat_hbm.at[pl.ds(d, _CHUNK)], buf, sem)

        def wait(buf, sem):
            pltpu.make_async_copy(flat_hbm.at[pl.ds(0, _CHUNK)], buf,
                                  sem).wait()

        def process_chunk(c, buf):
            pos = pos0 + c * _CHUNK
            d = jnp.minimum(pos, total_tok - _CHUNK)

            def seg_body(i, _):
                s_lo = extract(i)
                s_hi = extract(i + 1)
                r_lo = jnp.clip(jnp.maximum(s_lo, pos) - d, 0, _CHUNK)
                r_hi = jnp.clip(jnp.minimum(s_hi, pos + _CHUNK) - d,
                                0, _CHUNK)

                @pl.when(r_hi > r_lo)
                def _():
                    acc0 = tuple(stage[i, pl.ds(k * _LANES, _LANES)]
                                 for k in range(groups))

                    def row_body(r, acc):
                        return tuple(
                            acc[k] + buf[r, pl.ds(k * _LANES, _LANES)]
                            for k in range(groups))

                    acc = lax.fori_loop(r_lo, r_hi, row_body, acc0)
                    for k in range(groups):
                        stage[i, pl.ds(k * _LANES, _LANES)] = acc[k]

                return 0

            lax.fori_loop(0, _SEGS_PER_WORKER, seg_body, 0)

        issue(0, buf_a, sem_a)

        def pair_body(c2, _):
            c = 2 * c2
            wait(buf_a, sem_a)
            issue(c + 1, buf_b, sem_b)
            process_chunk(c, buf_a)
            wait(buf_b, sem_b)
            issue(c + 2, buf_a, sem_a)
            process_chunk(c + 1, buf_b)
            return 0

        lax.fori_loop(0, npair, pair_body, 0)
        wait(buf_a, sem_a)

        pltpu.sync_copy(stage, out_hbm.at[pl.ds(seg_base, _SEGS_PER_WORKER)])

    return pool(flat, cu_pad)


def _head_body(sums_ref, inv_ref, we_ref, wc_ref, out_ref):
    mean = sums_ref[...] * inv_ref[...]
    combined = jax.lax.dot_general(
        wc_ref[...], we_ref[...], (((1,), (0,)), ((), ())),
        precision=jax.lax.Precision.HIGHEST,
        preferred_element_type=jnp.float32)
    logits = jax.lax.dot_general(
        mean, combined, (((1,), (1,)), ((), ())),
        precision=jax.lax.Precision.HIGHEST,
        preferred_element_type=jnp.float32)
    m = jnp.max(logits, axis=-1, keepdims=True)
    sh = logits - m
    lse = jnp.log(jnp.sum(jnp.exp(sh), axis=-1, keepdims=True))
    out_ref[...] = sh - lse


def kernel(flat, cu_seqlens, W_e, W_c):
    total_tok, emb = flat.shape
    num_sents = cu_seqlens.shape[0] - 1
    ntags = W_c.shape[0]

    cu = cu_seqlens.astype(jnp.int32)
    inv = 1.0 / jnp.maximum(cu[1:] - cu[:-1], 1).astype(jnp.float32)
    inv = inv.reshape(num_sents, 1)
    cu_pad = jnp.full((num_sents + 2 * _SEGS_PER_WORKER,), total_tok,
                      jnp.int32).at[:num_sents + 1].set(cu)

    sums = _sc_pool(flat, cu_pad, total_tok, emb, num_sents)

    out = pl.pallas_call(
        _head_body,
        in_specs=[
            pl.BlockSpec((num_sents, emb), lambda: (0, 0)),
            pl.BlockSpec((num_sents, 1), lambda: (0, 0)),
            pl.BlockSpec((emb, emb), lambda: (0, 0)),
            pl.BlockSpec((ntags, emb), lambda: (0, 0)),
        ],
        out_specs=pl.BlockSpec((num_sents, ntags), lambda: (0, 0)),
        out_shape=jax.ShapeDtypeStruct((num_sents, ntags), jnp.float32),
    )(sums, inv, W_e, W_c)
    return out
